# baseline (speedup 1.0000x reference)
import functools

import jax
import jax.numpy as jnp
from jax import lax
from jax.experimental import pallas as pl
from jax.experimental.pallas import tpu as pltpu
from jax.experimental.pallas import tpu_sc as plsc

_NC = 2
_NS = 16
_NW = _NC * _NS
_C = 64


_NBUF = 4


def _make_spmm(NP, D, E_pad):
    PW = E_pad // _NW
    K = PW // _C
    NG = K // _NBUF
    RP = NP // _NS
    mesh = plsc.VectorSubcoreMesh(core_axis_name="c", subcore_axis_name="s")

    @functools.partial(
        pl.kernel,
        out_type=jax.ShapeDtypeStruct((_NC, NP, D), jnp.float32),
        mesh=mesh,
        scratch_types=[
            pltpu.VMEM((2, 2, _NBUF, _C), jnp.int32),
            pltpu.VMEM((_NBUF, _C, D), jnp.float32),
            pltpu.VMEM((8, D), jnp.float32),
            pltpu.VMEM_SHARED((NP, D), jnp.float32),
            *([pltpu.SemaphoreType.DMA] * (2 * _NBUF + 2)),
        ],
        compiler_params=pltpu.CompilerParams(use_tc_tiling_on_sc=False),
    )
    def spmm(x_hbm, idx_hbm, out_hbm, idxb, gbuf, zbuf, acc, *sems):
        gsems, ssems, isems = sems[:_NBUF], sems[_NBUF:2 * _NBUF], sems[2 * _NBUF:]
        cid = lax.axis_index("c")
        sid = lax.axis_index("s")
        wid = sid * _NC + cid
        zv = jnp.zeros((16,), jnp.float32)
        for i in range(8):
            for j in range(D // 16):
                zbuf[i, pl.ds(j * 16, 16)] = zv
        for r in range(RP // 8):
            pltpu.sync_copy(zbuf, acc.at[pl.ds(sid * RP + r * 8, 8)])
        plsc.subcore_barrier()

        def idx_fetch(g, u):
            pltpu.async_copy(idx_hbm.at[wid, g], idxb.at[u], isems[u])

        def idx_wait(g, u):
            pltpu.make_async_copy(idx_hbm.at[wid, g], idxb.at[u],
                                  isems[u]).wait()

        def gather(u, b):
            pltpu.async_copy(x_hbm.at[idxb.at[u, 0, b]], gbuf.at[b], gsems[b])

        def gather_wait(u, b):
            pltpu.make_async_copy(x_hbm.at[idxb.at[u, 0, b]], gbuf.at[b],
                                  gsems[b]).wait()

        def scatter(u, b):
            pltpu.async_copy(gbuf.at[b], acc.at[idxb.at[u, 1, b]], ssems[b],
                             add=True)

        def scatter_wait(u, b):
            pltpu.make_async_copy(gbuf.at[b], acc.at[idxb.at[u, 1, b]],
                                  ssems[b]).wait()

        def run_group(u):
            for b in range(_NBUF):
                gather_wait(u, b)
                scatter(u, b)
            for b in range(_NBUF):
                scatter_wait(u, b)

        idx_fetch(0, 0)
        idx_wait(0, 0)
        idx_fetch(1, 1)
        for b in range(_NBUF):
            gather(0, b)

        def body(i, carry):
            run_group(0)
            idx_fetch(2 * i + 2, 0)
            idx_wait(2 * i + 1, 1)
            for b in range(_NBUF):
                gather(1, b)
            run_group(1)
            idx_fetch(2 * i + 3, 1)
            idx_wait(2 * i + 2, 0)
            for b in range(_NBUF):
                gather(0, b)
            return carry

        lax.fori_loop(0, NG // 2 - 1, body, 0)
        run_group(0)
        idx_wait(NG - 1, 1)
        for b in range(_NBUF):
            gather(1, b)
        run_group(1)

        plsc.subcore_barrier()
        pltpu.sync_copy(acc.at[pl.ds(sid * RP, RP)],
                        out_hbm.at[cid, pl.ds(sid * RP, RP)])

    return spmm


def _mid_body(s1_ref, w1_ref, w2_ref, g_ref, invd_ref, *, IN):
    a = s1_ref[0] + s1_ref[1]
    deg = a[:, IN]
    invd = 1.0 / jnp.maximum(deg, 1.0)
    ax = a[:, :IN] * invd[:, None]
    h = jnp.maximum(jnp.dot(ax, w1_ref[...], preferred_element_type=jnp.float32), 0.0)
    g = jnp.dot(h, w2_ref[...], preferred_element_type=jnp.float32)
    g_ref[...] = g
    invd_ref[...] = jnp.broadcast_to(invd[:, None], invd_ref.shape)


def _fin_body(s2_ref, invd_ref, o_ref):
    o_ref[...] = (s2_ref[0] + s2_ref[1]) * invd_ref[...]


def kernel(x, edge_index, W1, W2):
    N, IN = x.shape
    H = W1.shape[1]
    E = edge_index.shape[1]
    D1 = IN + 16
    NP = 10112
    Q = _NW * _C * _NBUF * 2
    E_pad = -(-E // Q) * Q
    PW = E_pad // _NW
    K = PW // _C
    NG = K // _NBUF

    row = edge_index[0].astype(jnp.int32)
    col = edge_index[1].astype(jnp.int32)
    padi = jnp.full((E_pad - E,), N, jnp.int32)
    colp = jnp.concatenate([col, padi]).reshape(_NW, NG, _NBUF, _C)
    rowp = jnp.concatenate([row, padi]).reshape(_NW, NG, _NBUF, _C)
    idxp = jnp.stack([colp, rowp], axis=2)

    x_aug = jnp.zeros((NP, D1), jnp.float32)
    x_aug = x_aug.at[:N, :IN].set(x).at[:N, IN].set(1.0)

    spmm1 = _make_spmm(NP, D1, E_pad)
    spmm2 = _make_spmm(NP, IN, E_pad)

    s1 = spmm1(x_aug, idxp)

    BN = 1264
    grid = (NP // BN,)
    g, invd = pl.pallas_call(
        functools.partial(_mid_body, IN=IN),
        grid=grid,
        in_specs=[
            pl.BlockSpec((_NC, BN, D1), lambda i: (0, i, 0)),
            pl.BlockSpec((IN, H), lambda i: (0, 0)),
            pl.BlockSpec((H, IN), lambda i: (0, 0)),
        ],
        out_specs=[
            pl.BlockSpec((BN, IN), lambda i: (i, 0)),
            pl.BlockSpec((BN, IN), lambda i: (i, 0)),
        ],
        out_shape=[
            jax.ShapeDtypeStruct((NP, IN), jnp.float32),
            jax.ShapeDtypeStruct((NP, IN), jnp.float32),
        ],
    )(s1, W1, W2)

    s2 = spmm2(g, idxp)

    o = pl.pallas_call(
        _fin_body,
        grid=grid,
        in_specs=[
            pl.BlockSpec((_NC, BN, IN), lambda i: (0, i, 0)),
            pl.BlockSpec((BN, IN), lambda i: (i, 0)),
        ],
        out_specs=pl.BlockSpec((BN, IN), lambda i: (i, 0)),
        out_shape=jax.ShapeDtypeStruct((NP, IN), jnp.float32),
    )(s2, invd)
    return o[:N]


# speedup vs baseline: 2.5390x; 2.5390x over previous
import functools

import jax
import jax.numpy as jnp
from jax import lax
from jax.experimental import pallas as pl
from jax.experimental.pallas import tpu as pltpu
from jax.experimental.pallas import tpu_sc as plsc

_NC = 2
_NS = 16
_NW = _NC * _NS
_C = 64


_NBUF = 4


def _make_spmm(NP, D, E_pad):
    PW = E_pad // _NW
    K = PW // _C
    NG = K // _NBUF
    RP = NP // _NS
    mesh = plsc.VectorSubcoreMesh(core_axis_name="c", subcore_axis_name="s")

    @functools.partial(
        pl.kernel,
        out_type=jax.ShapeDtypeStruct((_NC, NP, D), jnp.float32),
        mesh=mesh,
        scratch_types=[
            pltpu.VMEM((2, 2, _NBUF, _C), jnp.int32),
            pltpu.VMEM((_NBUF, _C, D), jnp.float32),
            pltpu.VMEM((8, D), jnp.float32),
            pltpu.VMEM_SHARED((NP, D), jnp.float32),
            *([pltpu.SemaphoreType.DMA] * (2 * _NBUF + 2)),
        ],
        compiler_params=pltpu.CompilerParams(use_tc_tiling_on_sc=False),
    )
    def spmm(x_hbm, idx_hbm, out_hbm, idxb, gbuf, zbuf, acc, *sems):
        gsems, ssems, isems = sems[:_NBUF], sems[_NBUF:2 * _NBUF], sems[2 * _NBUF:]
        cid = lax.axis_index("c")
        sid = lax.axis_index("s")
        wid = sid * _NC + cid
        zv = jnp.zeros((16,), jnp.float32)
        for i in range(8):
            for j in range(D // 16):
                zbuf[i, pl.ds(j * 16, 16)] = zv
        for r in range(RP // 8):
            pltpu.sync_copy(zbuf, acc.at[pl.ds(sid * RP + r * 8, 8)])
        plsc.subcore_barrier()

        def idx_fetch(g, u):
            pltpu.async_copy(idx_hbm.at[wid, g], idxb.at[u], isems[u])

        def idx_wait(g, u):
            pltpu.make_async_copy(idx_hbm.at[wid, g], idxb.at[u],
                                  isems[u]).wait()

        def gather(u, b):
            pltpu.async_copy(x_hbm.at[idxb.at[u, 0, b]], gbuf.at[b], gsems[b])

        def gather_wait(u, b):
            pltpu.make_async_copy(x_hbm.at[idxb.at[u, 0, b]], gbuf.at[b],
                                  gsems[b]).wait()

        def scatter(u, b):
            pltpu.async_copy(gbuf.at[b], acc.at[idxb.at[u, 1, b]], ssems[b],
                             add=True)

        def scatter_wait(u, b):
            pltpu.make_async_copy(gbuf.at[b], acc.at[idxb.at[u, 1, b]],
                                  ssems[b]).wait()

        def run_group(u):
            for b in range(_NBUF):
                gather_wait(u, b)
                scatter(u, b)
            for b in range(_NBUF):
                scatter_wait(u, b)

        idx_fetch(0, 0)
        idx_wait(0, 0)
        idx_fetch(1, 1)
        for b in range(_NBUF):
            gather(0, b)

        def body(i, carry):
            run_group(0)
            idx_fetch(2 * i + 2, 0)
            idx_wait(2 * i + 1, 1)
            for b in range(_NBUF):
                gather(1, b)
            run_group(1)
            idx_fetch(2 * i + 3, 1)
            idx_wait(2 * i + 2, 0)
            for b in range(_NBUF):
                gather(0, b)
            return carry

        lax.fori_loop(0, NG // 2 - 1, body, 0)
        run_group(0)
        idx_wait(NG - 1, 1)
        for b in range(_NBUF):
            gather(1, b)
        run_group(1)

        plsc.subcore_barrier()
        pltpu.sync_copy(acc.at[pl.ds(sid * RP, RP)],
                        out_hbm.at[cid, pl.ds(sid * RP, RP)])

    return spmm


def _mid_body(s1_ref, w1_ref, w2_ref, g_ref, invd_ref, *, IN):
    a = s1_ref[0] + s1_ref[1]
    deg = a[:, IN]
    invd = 1.0 / jnp.maximum(deg, 1.0)
    ax = a[:, :IN] * invd[:, None]
    h = jnp.maximum(jnp.dot(ax, w1_ref[...], preferred_element_type=jnp.float32), 0.0)
    g = jnp.dot(h, w2_ref[...], preferred_element_type=jnp.float32)
    g_ref[...] = g
    invd_ref[...] = jnp.broadcast_to(invd[:, None], invd_ref.shape)


def _fin_body(s2_ref, invd_ref, o_ref):
    o_ref[...] = (s2_ref[0] + s2_ref[1]) * invd_ref[...]


def kernel(x, edge_index, W1, W2):
    N, IN = x.shape
    H = W1.shape[1]
    E = edge_index.shape[1]
    D1 = IN + 16
    NP = 10112
    Q = _NW * _C * _NBUF * 2
    E_pad = -(-E // Q) * Q
    PW = E_pad // _NW
    K = PW // _C
    NG = K // _NBUF

    row = edge_index[0].astype(jnp.int32)
    col = edge_index[1].astype(jnp.int32)
    padi = N + (jnp.arange(E_pad - E, dtype=jnp.int32) % (NP - N))
    colp = jnp.concatenate([col, padi]).reshape(_NW, NG, _NBUF, _C)
    rowp = jnp.concatenate([row, padi]).reshape(_NW, NG, _NBUF, _C)
    idxp = jnp.stack([colp, rowp], axis=2)

    x_aug = jnp.zeros((NP, D1), jnp.float32)
    x_aug = x_aug.at[:N, :IN].set(x).at[:N, IN].set(1.0)

    spmm1 = _make_spmm(NP, D1, E_pad)
    spmm2 = _make_spmm(NP, IN, E_pad)

    s1 = spmm1(x_aug, idxp)

    BN = 1264
    grid = (NP // BN,)
    g, invd = pl.pallas_call(
        functools.partial(_mid_body, IN=IN),
        grid=grid,
        in_specs=[
            pl.BlockSpec((_NC, BN, D1), lambda i: (0, i, 0)),
            pl.BlockSpec((IN, H), lambda i: (0, 0)),
            pl.BlockSpec((H, IN), lambda i: (0, 0)),
        ],
        out_specs=[
            pl.BlockSpec((BN, IN), lambda i: (i, 0)),
            pl.BlockSpec((BN, IN), lambda i: (i, 0)),
        ],
        out_shape=[
            jax.ShapeDtypeStruct((NP, IN), jnp.float32),
            jax.ShapeDtypeStruct((NP, IN), jnp.float32),
        ],
    )(s1, W1, W2)

    s2 = spmm2(g, idxp)

    o = pl.pallas_call(
        _fin_body,
        grid=grid,
        in_specs=[
            pl.BlockSpec((_NC, BN, IN), lambda i: (0, i, 0)),
            pl.BlockSpec((BN, IN), lambda i: (i, 0)),
        ],
        out_specs=pl.BlockSpec((BN, IN), lambda i: (i, 0)),
        out_shape=jax.ShapeDtypeStruct((NP, IN), jnp.float32),
    )(s2, invd)
    return o[:N]


# speedup vs baseline: 2.5709x; 1.0126x over previous
import functools

import jax
import jax.numpy as jnp
from jax import lax
from jax.experimental import pallas as pl
from jax.experimental.pallas import tpu as pltpu
from jax.experimental.pallas import tpu_sc as plsc

_NC = 2
_NS = 16
_NW = _NC * _NS
_C = 128


_NBUF = 2


def _make_spmm(NP, D, E_pad):
    PW = E_pad // _NW
    K = PW // _C
    NG = K // _NBUF
    RP = NP // _NS
    mesh = plsc.VectorSubcoreMesh(core_axis_name="c", subcore_axis_name="s")

    @functools.partial(
        pl.kernel,
        out_type=jax.ShapeDtypeStruct((_NC, NP, D), jnp.float32),
        mesh=mesh,
        scratch_types=[
            pltpu.VMEM((2, 2, _NBUF, _C), jnp.int32),
            pltpu.VMEM((_NBUF, _C, D), jnp.float32),
            pltpu.VMEM((8, D), jnp.float32),
            pltpu.VMEM_SHARED((NP, D), jnp.float32),
            *([pltpu.SemaphoreType.DMA] * (2 * _NBUF + 2)),
        ],
        compiler_params=pltpu.CompilerParams(use_tc_tiling_on_sc=False),
    )
    def spmm(x_hbm, idx_hbm, out_hbm, idxb, gbuf, zbuf, acc, *sems):
        gsems, ssems, isems = sems[:_NBUF], sems[_NBUF:2 * _NBUF], sems[2 * _NBUF:]
        cid = lax.axis_index("c")
        sid = lax.axis_index("s")
        wid = sid * _NC + cid
        zv = jnp.zeros((16,), jnp.float32)
        for i in range(8):
            for j in range(D // 16):
                zbuf[i, pl.ds(j * 16, 16)] = zv
        for r in range(RP // 8):
            pltpu.sync_copy(zbuf, acc.at[pl.ds(sid * RP + r * 8, 8)])
        plsc.subcore_barrier()

        def idx_fetch(g, u):
            pltpu.async_copy(idx_hbm.at[wid, g], idxb.at[u], isems[u])

        def idx_wait(g, u):
            pltpu.make_async_copy(idx_hbm.at[wid, g], idxb.at[u],
                                  isems[u]).wait()

        def gather(u, b):
            pltpu.async_copy(x_hbm.at[idxb.at[u, 0, b]], gbuf.at[b], gsems[b])

        def gather_wait(u, b):
            pltpu.make_async_copy(x_hbm.at[idxb.at[u, 0, b]], gbuf.at[b],
                                  gsems[b]).wait()

        def scatter(u, b):
            pltpu.async_copy(gbuf.at[b], acc.at[idxb.at[u, 1, b]], ssems[b],
                             add=True)

        def scatter_wait(u, b):
            pltpu.make_async_copy(gbuf.at[b], acc.at[idxb.at[u, 1, b]],
                                  ssems[b]).wait()

        def run_group(u):
            for b in range(_NBUF):
                gather_wait(u, b)
                scatter(u, b)
            for b in range(_NBUF):
                scatter_wait(u, b)

        idx_fetch(0, 0)
        idx_wait(0, 0)
        idx_fetch(1, 1)
        for b in range(_NBUF):
            gather(0, b)

        def body(i, carry):
            run_group(0)
            idx_fetch(2 * i + 2, 0)
            idx_wait(2 * i + 1, 1)
            for b in range(_NBUF):
                gather(1, b)
            run_group(1)
            idx_fetch(2 * i + 3, 1)
            idx_wait(2 * i + 2, 0)
            for b in range(_NBUF):
                gather(0, b)
            return carry

        lax.fori_loop(0, NG // 2 - 1, body, 0)
        run_group(0)
        idx_wait(NG - 1, 1)
        for b in range(_NBUF):
            gather(1, b)
        run_group(1)

        plsc.subcore_barrier()
        pltpu.sync_copy(acc.at[pl.ds(sid * RP, RP)],
                        out_hbm.at[cid, pl.ds(sid * RP, RP)])

    return spmm


def _mid_body(s1_ref, w1_ref, w2_ref, g_ref, invd_ref, *, IN):
    a = s1_ref[0] + s1_ref[1]
    deg = a[:, IN]
    invd = 1.0 / jnp.maximum(deg, 1.0)
    ax = a[:, :IN] * invd[:, None]
    h = jnp.maximum(jnp.dot(ax, w1_ref[...], preferred_element_type=jnp.float32), 0.0)
    g = jnp.dot(h, w2_ref[...], preferred_element_type=jnp.float32)
    g_ref[...] = g
    invd_ref[...] = jnp.broadcast_to(invd[:, None], invd_ref.shape)


def _fin_body(s2_ref, invd_ref, o_ref):
    o_ref[...] = (s2_ref[0] + s2_ref[1]) * invd_ref[...]


def kernel(x, edge_index, W1, W2):
    N, IN = x.shape
    H = W1.shape[1]
    E = edge_index.shape[1]
    D1 = IN + 16
    NP = 10112
    Q = _NW * _C * _NBUF * 2
    E_pad = -(-E // Q) * Q
    PW = E_pad // _NW
    K = PW // _C
    NG = K // _NBUF

    row = edge_index[0].astype(jnp.int32)
    col = edge_index[1].astype(jnp.int32)
    padi = N + (jnp.arange(E_pad - E, dtype=jnp.int32) % (NP - N))
    colp = jnp.concatenate([col, padi]).reshape(_NW, NG, _NBUF, _C)
    rowp = jnp.concatenate([row, padi]).reshape(_NW, NG, _NBUF, _C)
    idxp = jnp.stack([colp, rowp], axis=2)

    x_aug = jnp.zeros((NP, D1), jnp.float32)
    x_aug = x_aug.at[:N, :IN].set(x).at[:N, IN].set(1.0)

    spmm1 = _make_spmm(NP, D1, E_pad)
    spmm2 = _make_spmm(NP, IN, E_pad)

    s1 = spmm1(x_aug, idxp)

    BN = 1264
    grid = (NP // BN,)
    g, invd = pl.pallas_call(
        functools.partial(_mid_body, IN=IN),
        grid=grid,
        in_specs=[
            pl.BlockSpec((_NC, BN, D1), lambda i: (0, i, 0)),
            pl.BlockSpec((IN, H), lambda i: (0, 0)),
            pl.BlockSpec((H, IN), lambda i: (0, 0)),
        ],
        out_specs=[
            pl.BlockSpec((BN, IN), lambda i: (i, 0)),
            pl.BlockSpec((BN, IN), lambda i: (i, 0)),
        ],
        out_shape=[
            jax.ShapeDtypeStruct((NP, IN), jnp.float32),
            jax.ShapeDtypeStruct((NP, IN), jnp.float32),
        ],
    )(s1, W1, W2)

    s2 = spmm2(g, idxp)

    o = pl.pallas_call(
        _fin_body,
        grid=grid,
        in_specs=[
            pl.BlockSpec((_NC, BN, IN), lambda i: (0, i, 0)),
            pl.BlockSpec((BN, IN), lambda i: (i, 0)),
        ],
        out_specs=pl.BlockSpec((BN, IN), lambda i: (i, 0)),
        out_shape=jax.ShapeDtypeStruct((NP, IN), jnp.float32),
    )(s2, invd)
    return o[:N]
